# baseline (speedup 1.0000x reference)
import functools

import jax
import jax.numpy as jnp
from jax import lax
from jax.experimental import pallas as pl
from jax.experimental.pallas import tpu as pltpu
from jax.experimental.pallas import tpu_sc as plsc

_N = 2097152
_E = _N * 4
_NW = 32
_RPW = _N // _NW
_CHR = 4096
_CHE = _CHR * 4
_CHUNKS = _RPW // _CHR
_GROUPS = _CHR // 16


def _sc_kernel(o_hbm, t_hbm, l_hbm, tot_hbm, cnt_hbm,
               o_bufs, t_bufs, l_bufs, sems, stage):
    wid = lax.axis_index("s") * 2 + lax.axis_index("c")
    row0 = wid * _RPW

    def start(c, b):
        r0 = row0 + c * _CHR
        pltpu.async_copy(o_hbm.at[pl.ds(r0 * 4, _CHE)], o_bufs[b], sems[b])
        pltpu.async_copy(t_hbm.at[pl.ds(r0 * 4, _CHE)], t_bufs[b], sems[b])
        pltpu.async_copy(l_hbm.at[pl.ds(r0, _CHR)], l_bufs[b], sems[b])

    def wait(c, b):
        r0 = row0 + c * _CHR
        pltpu.make_async_copy(
            o_hbm.at[pl.ds(r0 * 4, _CHE)], o_bufs[b], sems[b]
        ).wait()
        pltpu.make_async_copy(
            t_hbm.at[pl.ds(r0 * 4, _CHE)], t_bufs[b], sems[b]
        ).wait()
        pltpu.make_async_copy(
            l_hbm.at[pl.ds(r0, _CHR)], l_bufs[b], sems[b]
        ).wait()

    def compute(b, acc, cnt):
        o_buf, t_buf, l_buf = o_bufs[b], t_bufs[b], l_bufs[b]

        @plsc.parallel_loop(0, _GROUPS, 1, unroll=8, carry=(acc, cnt))
        def group_body(g, carry2):
            acc2, cnt2 = carry2
            l16 = l_buf[pl.ds(g * 16, 16)]
            msk = jnp.where(l16 == 1, 1.0, 0.0)
            off = (g >> 3) * 512 + (g & 7) * 16
            hsum = jnp.zeros((16,), jnp.float32)
            for col in range(4):
                o_v = o_buf[pl.ds(off + col * 128, 16)]
                t_v = t_buf[pl.ds(off + col * 128, 16)]
                d = o_v - t_v
                a = jnp.abs(d)
                q = jnp.minimum(a, 1.0)
                hsum = hsum + q * (a - 0.5 * q)
            acc2 = acc2 + msk * hsum
            cnt2 = cnt2 + msk
            return acc2, cnt2

        return group_body

    start(0, 0)
    zeros = jnp.zeros((16,), jnp.float32)

    def pair_body(i, carry):
        acc, cnt = carry
        c0 = i * 2
        start(c0 + 1, 1)
        wait(c0, 0)
        acc, cnt = compute(0, acc, cnt)

        @pl.when(c0 + 2 < _CHUNKS)
        def _():
            start(c0 + 2, 0)

        wait(c0 + 1, 1)
        return compute(1, acc, cnt)

    acc, cnt = lax.fori_loop(0, _CHUNKS // 2, pair_body, (zeros, zeros))

    stage[...] = acc
    pltpu.sync_copy(stage, tot_hbm.at[wid])
    stage[...] = cnt
    pltpu.sync_copy(stage, cnt_hbm.at[wid])


def _finish_kernel(tot_ref, cnt_ref, out_ref):
    total = jnp.sum(tot_ref[...])
    npos = jnp.sum(cnt_ref[...])
    out_ref[0] = total / jnp.maximum(npos * 4.0, 1.0)


@jax.jit
def kernel(out_bbox, labels, bbox_targets):
    o_flat = out_bbox.reshape(_N // 128, 128, 4).transpose(0, 2, 1).reshape(_E)
    t_flat = (
        bbox_targets.reshape(_N // 128, 128, 4).transpose(0, 2, 1).reshape(_E)
    )

    mesh = plsc.VectorSubcoreMesh(core_axis_name="c", subcore_axis_name="s")
    sc = pl.kernel(
        _sc_kernel,
        out_type=[
            jax.ShapeDtypeStruct((_NW, 16), jnp.float32),
            jax.ShapeDtypeStruct((_NW, 16), jnp.float32),
        ],
        mesh=mesh,
        scratch_types=[
            [pltpu.VMEM((_CHE,), jnp.float32)] * 2,
            [pltpu.VMEM((_CHE,), jnp.float32)] * 2,
            [pltpu.VMEM((_CHR,), jnp.int32)] * 2,
            [pltpu.SemaphoreType.DMA] * 2,
            pltpu.VMEM((16,), jnp.float32),
        ],
        compiler_params=pltpu.CompilerParams(
            use_tc_tiling_on_sc=False, needs_layout_passes=False
        ),
    )
    tot, cnt = sc(o_flat, t_flat, labels)

    out = pl.pallas_call(
        _finish_kernel,
        out_specs=pl.BlockSpec(memory_space=pltpu.SMEM),
        out_shape=jax.ShapeDtypeStruct((1,), jnp.float32),
    )(tot, cnt)
    return out[0]


# speedup vs baseline: 1.6550x; 1.6550x over previous
import functools

import jax
import jax.numpy as jnp
from jax import lax
from jax.experimental import pallas as pl
from jax.experimental.pallas import tpu as pltpu
from jax.experimental.pallas import tpu_sc as plsc

_N = 2097152
_E = _N * 4
_NW = 32
_RPW = _N // _NW
_CHR = 4096
_CHE = _CHR * 4
_CHUNKS = _RPW // _CHR
_GROUPS = _CHR // 16


def _sc_kernel(o_hbm, t_hbm, l_hbm, tot_hbm, cnt_hbm,
               o_bufs, t_bufs, l_bufs, sems, stage):
    wid = lax.axis_index("s") * 2 + lax.axis_index("c")
    row0 = wid * _RPW

    def start(c, b):
        r0 = row0 + c * _CHR
        pltpu.async_copy(o_hbm.at[pl.ds(r0 * 4, _CHE)], o_bufs[b], sems[b])
        pltpu.async_copy(t_hbm.at[pl.ds(r0 * 4, _CHE)], t_bufs[b], sems[b])
        pltpu.async_copy(l_hbm.at[pl.ds(r0, _CHR)], l_bufs[b], sems[b])

    def wait(c, b):
        r0 = row0 + c * _CHR
        pltpu.make_async_copy(
            o_hbm.at[pl.ds(r0 * 4, _CHE)], o_bufs[b], sems[b]
        ).wait()
        pltpu.make_async_copy(
            t_hbm.at[pl.ds(r0 * 4, _CHE)], t_bufs[b], sems[b]
        ).wait()
        pltpu.make_async_copy(
            l_hbm.at[pl.ds(r0, _CHR)], l_bufs[b], sems[b]
        ).wait()

    def compute(b, acc, cnt):
        o_buf, t_buf, l_buf = o_bufs[b], t_bufs[b], l_bufs[b]

        @plsc.parallel_loop(0, _GROUPS, 1, unroll=8, carry=(acc, cnt))
        def group_body(g, carry2):
            acc2, cnt2 = carry2
            l16 = l_buf[pl.ds(g * 16, 16)]
            msk = jnp.where(l16 == 1, 1.0, 0.0)
            off = (g >> 3) * 512 + (g & 7) * 16
            hsum = jnp.zeros((16,), jnp.float32)
            for col in range(4):
                o_v = o_buf[pl.ds(off + col * 128, 16)]
                t_v = t_buf[pl.ds(off + col * 128, 16)]
                d = o_v - t_v
                a = jnp.abs(d)
                q = jnp.minimum(a, 1.0)
                hsum = hsum + q * (a - 0.5 * q)
            acc2 = acc2 + msk * hsum
            cnt2 = cnt2 + msk
            return acc2, cnt2

        return group_body

    start(0, 0)
    zeros = jnp.zeros((16,), jnp.float32)

    def pair_body(i, carry):
        acc, cnt = carry
        c0 = i * 2
        start(c0 + 1, 1)
        wait(c0, 0)
        acc, cnt = compute(0, acc, cnt)

        @pl.when(c0 + 2 < _CHUNKS)
        def _():
            start(c0 + 2, 0)

        wait(c0 + 1, 1)
        return compute(1, acc, cnt)

    acc, cnt = lax.fori_loop(0, _CHUNKS // 2, pair_body, (zeros, zeros))

    stage[...] = acc
    pltpu.sync_copy(stage, tot_hbm.at[wid])
    stage[...] = cnt
    pltpu.sync_copy(stage, cnt_hbm.at[wid])


def _finish_kernel(tot_ref, cnt_ref, out_ref):
    total = jnp.sum(tot_ref[...])
    npos = jnp.sum(cnt_ref[...])
    out_ref[0] = total / jnp.maximum(npos * 4.0, 1.0)


_TCX = 8192
_TCB = 512
_TCGRID = _TCX // _TCB


def _tc_kernel(o_ref, t_ref, l_ref, out_ref, acc_ref):
    step = pl.program_id(0)

    @pl.when(step == 0)
    def _init():
        acc_ref[0] = 0.0
        acc_ref[1] = 0.0

    diff = o_ref[...] - t_ref[...]
    a = jnp.abs(diff)
    q = jnp.minimum(a, 1.0)
    h = q * (a - 0.5 * q)

    mf = (l_ref[...] == 1).astype(jnp.float32)
    m3 = mf.reshape(_TCB, 2, 128)
    m8 = jnp.broadcast_to(m3[:, :, None, :], (_TCB, 2, 4, 128)).reshape(
        _TCB, 8, 128
    )
    acc_ref[0] += jnp.sum(h * m8)
    acc_ref[1] += jnp.sum(mf)

    @pl.when(step == _TCGRID - 1)
    def _fini():
        denom = jnp.maximum(acc_ref[1] * 4.0, 1.0)
        out_ref[0] = acc_ref[0] / denom


@jax.jit
def kernel(out_bbox, labels, bbox_targets):
    o3 = (
        out_bbox.reshape(_N // 128, 128, 4)
        .transpose(0, 2, 1)
        .reshape(_TCX, 8, 128)
    )
    t3 = (
        bbox_targets.reshape(_N // 128, 128, 4)
        .transpose(0, 2, 1)
        .reshape(_TCX, 8, 128)
    )
    l2 = labels.reshape(_N // 128, 128)

    out = pl.pallas_call(
        _tc_kernel,
        grid=(_TCGRID,),
        in_specs=[
            pl.BlockSpec((_TCB, 8, 128), lambda i: (i, 0, 0)),
            pl.BlockSpec((_TCB, 8, 128), lambda i: (i, 0, 0)),
            pl.BlockSpec((_TCB * 2, 128), lambda i: (i, 0)),
        ],
        out_specs=pl.BlockSpec(memory_space=pltpu.SMEM),
        out_shape=jax.ShapeDtypeStruct((1,), jnp.float32),
        scratch_shapes=[pltpu.SMEM((2,), jnp.float32)],
    )(o3, t3, l2)
    return out[0]


@jax.jit
def _kernel_sc_only(out_bbox, labels, bbox_targets):
    o_flat = out_bbox.reshape(_N // 128, 128, 4).transpose(0, 2, 1).reshape(_E)
    t_flat = (
        bbox_targets.reshape(_N // 128, 128, 4).transpose(0, 2, 1).reshape(_E)
    )

    mesh = plsc.VectorSubcoreMesh(core_axis_name="c", subcore_axis_name="s")
    sc = pl.kernel(
        _sc_kernel,
        out_type=[
            jax.ShapeDtypeStruct((_NW, 16), jnp.float32),
            jax.ShapeDtypeStruct((_NW, 16), jnp.float32),
        ],
        mesh=mesh,
        scratch_types=[
            [pltpu.VMEM((_CHE,), jnp.float32)] * 2,
            [pltpu.VMEM((_CHE,), jnp.float32)] * 2,
            [pltpu.VMEM((_CHR,), jnp.int32)] * 2,
            [pltpu.SemaphoreType.DMA] * 2,
            pltpu.VMEM((16,), jnp.float32),
        ],
        compiler_params=pltpu.CompilerParams(
            use_tc_tiling_on_sc=False, needs_layout_passes=False
        ),
    )
    tot, cnt = sc(o_flat, t_flat, labels)

    return tot[0, 0]
